# Initial kernel scaffold; baseline (speedup 1.0000x reference)
import jax
import jax.numpy as jnp
from jax.experimental import pallas as pl


def kernel(beta, particle_id):
    raise NotImplementedError("write your pallas kernel here")


# speedup vs baseline: 4.8236x; 4.8236x over previous
import functools

import jax
import jax.numpy as jnp
from jax import lax
from jax.experimental import pallas as pl
from jax.experimental.pallas import tpu as pltpu
from jax.experimental.pallas import tpu_sc as plsc

N = 100000
NBINS = 512
NW = 32
CHUNK = 3136
PAD = NW * CHUNK - N
NVEC = CHUNK // 16
ROW = NBINS + 32

_mesh = plsc.VectorSubcoreMesh(core_axis_name="c", subcore_axis_name="s")


@functools.partial(
    pl.kernel,
    out_type=jax.ShapeDtypeStruct((NW, ROW), jnp.float32),
    mesh=_mesh,
    scratch_types=[
        pltpu.VMEM((CHUNK,), jnp.float32),
        pltpu.VMEM((CHUNK,), jnp.int32),
        pltpu.VMEM((16 * NBINS,), jnp.float32),
        pltpu.VMEM((ROW,), jnp.float32),
    ],
    compiler_params=pltpu.CompilerParams(
        use_tc_tiling_on_sc=False, needs_layout_passes=False
    ),
)
def _sc_partials(beta_hbm, pid_hbm, out_hbm, beta_v, pid_v, acc, res):
    wid = lax.axis_index("s") * 2 + lax.axis_index("c")
    base = wid * CHUNK
    pltpu.sync_copy(beta_hbm.at[pl.ds(base, CHUNK)], beta_v)
    pltpu.sync_copy(pid_hbm.at[pl.ds(base, CHUNK)], pid_v)

    neg = jnp.full((16,), -1.0, jnp.float32)

    def init_body(c, _):
        acc[pl.ds(c * 16, 16)] = neg
        return 0

    lax.fori_loop(0, NBINS, init_body, 0)

    loff = lax.iota(jnp.int32, 16) * NBINS
    zf = jnp.zeros((16,), jnp.float32)

    def body(i, carry):
        ns, nc = carry
        pv = pid_v[pl.ds(i * 16, 16)]
        bv = beta_v[pl.ds(i * 16, 16)]
        idx = loff + pv
        cur = plsc.load_gather(acc, [idx])
        plsc.store_scatter(acc, [idx], jnp.maximum(cur, bv))
        m = pv == 0
        ns = ns + jnp.where(m, bv, 0.0)
        nc = nc + jnp.where(m, 1.0, 0.0)
        return ns, nc

    ns, nc = lax.fori_loop(0, NVEC, body, (zf, zf))

    def red_body(c, _):
        m = acc[pl.ds(c * 16, 16)]
        for l in range(1, 16):
            m = jnp.maximum(m, acc[pl.ds(l * NBINS + c * 16, 16)])
        res[pl.ds(c * 16, 16)] = m
        return 0

    lax.fori_loop(0, NBINS // 16, red_body, 0)
    res[pl.ds(NBINS, 16)] = ns
    res[pl.ds(NBINS + 16, 16)] = nc
    pltpu.sync_copy(res, out_hbm.at[wid])


def _combine_body(x_ref, o_ref):
    x = x_ref[:, :]
    col = lax.broadcasted_iota(jnp.int32, (1, ROW), 1)
    m = jnp.max(x, axis=0, keepdims=True)
    present = (m >= 0.0) & (col >= 1) & (col < NBINS)
    bsum = jnp.sum(jnp.where(present, 1.0 - m, 0.0))
    npres = jnp.sum(present.astype(jnp.float32))
    colx = lax.broadcasted_iota(jnp.int32, (NW, ROW), 1)
    ns = jnp.sum(jnp.where((colx >= NBINS) & (colx < NBINS + 16), x, 0.0))
    nc = jnp.sum(jnp.where(colx >= NBINS + 16, x, 0.0)) - float(PAD)
    loss = bsum / npres
    noise_mean = ns / jnp.maximum(nc, 1.0)
    loss = jnp.where(nc > 0.5, loss + 0.1 * noise_mean, loss)
    o_ref[0, 0] = loss


_combine = pl.pallas_call(
    _combine_body,
    out_shape=jax.ShapeDtypeStruct((1, 1), jnp.float32),
    out_specs=pl.BlockSpec(memory_space=pltpu.SMEM),
)


def kernel(beta, particle_id):
    beta_p = jnp.pad(beta, (0, PAD))
    pid_p = jnp.pad(particle_id, (0, PAD))
    parts = _sc_partials(beta_p, pid_p)
    return _combine(parts)[0, 0]
